# Initial kernel scaffold; baseline (speedup 1.0000x reference)
import functools

import jax
import jax.numpy as jnp
from jax import lax
from jax.experimental import pallas as pl
from jax.experimental.pallas import tpu as pltpu
from jax.experimental.pallas import tpu_sc as plsc

N = 10000
E = 640000
D = 128
NTILES = 16
CHUNK = 128
NCHUNK = 320
EPAD = NTILES * NCHUNK * CHUNK
NROWS = 10240
ROWS_PER_TILE_ZERO = NROWS // NTILES
ROWS_PER_TILE_OUT = N // NTILES

_INV_SQRT2 = 0.7071067811865475


def _h2_body(x_ref, o_ref):
    x = x_ref[...]
    o_ref[...] = (x * x - 1.0) * _INV_SQRT2


def _h2(features):
    return pl.pallas_call(
        _h2_body,
        out_shape=jax.ShapeDtypeStruct((N, D), jnp.float32),
        grid=(10,),
        in_specs=[pl.BlockSpec((N // 10, D), lambda i: (i, 0))],
        out_specs=pl.BlockSpec((N // 10, D), lambda i: (i, 0)),
    )(features)


@functools.partial(
    pl.kernel,
    mesh=plsc.VectorSubcoreMesh(core_axis_name="c", subcore_axis_name="s"),
    out_type=[
        jax.ShapeDtypeStruct((N, D), jnp.float32),
        jax.ShapeDtypeStruct((N, D), jnp.float32),
    ],
    scratch_types=[
        pltpu.VMEM((NCHUNK, CHUNK), jnp.int32),
        pltpu.VMEM((NCHUNK, CHUNK), jnp.int32),
        pltpu.VMEM((CHUNK, D), jnp.float32),
        pltpu.VMEM_SHARED((NROWS, D), jnp.float32),
        pltpu.SemaphoreType.DMA,
    ],
)
def _sc_scatter(feat_hbm, h2_hbm, src_hbm, dst_hbm, zero_hbm,
                out1, out2, sidx, didx, rows, acc_sh, sem):
    c = lax.axis_index("c")
    s = lax.axis_index("s")

    pltpu.sync_copy(zero_hbm, rows)
    for k in range(ROWS_PER_TILE_ZERO // CHUNK):
        pltpu.sync_copy(
            rows, acc_sh.at[pl.ds(s * ROWS_PER_TILE_ZERO + k * CHUNK, CHUNK)])
    plsc.subcore_barrier()

    pltpu.sync_copy(src_hbm.at[s], sidx)
    pltpu.sync_copy(dst_hbm.at[s], didx)

    def body(j, carry):
        @pl.when(c == 0)
        def _():
            pltpu.async_copy(feat_hbm.at[sidx.at[j]], rows, sem).wait()

        @pl.when(c == 1)
        def _():
            pltpu.async_copy(h2_hbm.at[sidx.at[j]], rows, sem).wait()

        pltpu.sync_copy(rows, acc_sh.at[didx.at[j]], add=True)
        return carry

    lax.fori_loop(0, NCHUNK, body, 0)
    plsc.subcore_barrier()

    r0 = s * ROWS_PER_TILE_OUT

    @pl.when(c == 0)
    def _():
        pltpu.sync_copy(acc_sh.at[pl.ds(r0, ROWS_PER_TILE_OUT)],
                        out1.at[pl.ds(r0, ROWS_PER_TILE_OUT)])

    @pl.when(c == 1)
    def _():
        pltpu.sync_copy(acc_sh.at[pl.ds(r0, ROWS_PER_TILE_OUT)],
                        out2.at[pl.ds(r0, ROWS_PER_TILE_OUT)])


def kernel(features, edge_index):
    h2 = _h2(features)
    src = edge_index[0].astype(jnp.int32)
    dst = edge_index[1].astype(jnp.int32)
    pad = EPAD - E
    src_p = jnp.concatenate([src, jnp.zeros((pad,), jnp.int32)])
    dst_p = jnp.concatenate([dst, jnp.full((pad,), N, jnp.int32)])
    src3 = src_p.reshape(NTILES, NCHUNK, CHUNK)
    dst3 = dst_p.reshape(NTILES, NCHUNK, CHUNK)
    zero = jnp.zeros((CHUNK, D), jnp.float32)
    out1, out2 = _sc_scatter(features, h2, src3, dst3, zero)
    return jnp.concatenate([out1, out2], axis=-1)


# speedup vs baseline: 4.0618x; 4.0618x over previous
import functools

import jax
import jax.numpy as jnp
from jax import lax
from jax.experimental import pallas as pl
from jax.experimental.pallas import tpu as pltpu
from jax.experimental.pallas import tpu_sc as plsc

N = 10000
E = 640000
D = 128
NTILES = 16
CHUNK = 128
NCHUNK = 320
IDXB = 32
NSTAGE = NCHUNK // IDXB
EPAD = NTILES * NCHUNK * CHUNK
NROWS = 10240
ROWS_PER_TILE_ZERO = NROWS // NTILES
ROWS_PER_TILE_OUT = NROWS // NTILES

_INV_SQRT2 = 0.7071067811865475


def _h2_body(x_ref, o_ref):
    x = x_ref[...]
    o_ref[...] = (x * x - 1.0) * _INV_SQRT2


def _h2(features):
    return pl.pallas_call(
        _h2_body,
        out_shape=jax.ShapeDtypeStruct((N, D), jnp.float32),
        grid=(10,),
        in_specs=[pl.BlockSpec((N // 10, D), lambda i: (i, 0))],
        out_specs=pl.BlockSpec((N // 10, D), lambda i: (i, 0)),
    )(features)


@functools.partial(
    pl.kernel,
    mesh=plsc.VectorSubcoreMesh(core_axis_name="c", subcore_axis_name="s"),
    out_type=[
        jax.ShapeDtypeStruct((NROWS, D), jnp.float32),
        jax.ShapeDtypeStruct((NROWS, D), jnp.float32),
    ],
    scratch_types=[
        pltpu.VMEM((IDXB, CHUNK), jnp.int32),
        pltpu.VMEM((IDXB, CHUNK), jnp.int32),
        pltpu.VMEM((CHUNK, D), jnp.float32),
        pltpu.VMEM_SHARED((NROWS, D), jnp.float32),
        pltpu.SemaphoreType.DMA,
    ],
)
def _sc_scatter(feat_hbm, h2_hbm, src_hbm, dst_hbm, zero_hbm,
                out1, out2, sidx, didx, rows, acc_sh, sem):
    c = lax.axis_index("c")
    s = lax.axis_index("s")

    pltpu.sync_copy(zero_hbm, rows)
    for k in range(ROWS_PER_TILE_ZERO // CHUNK):
        pltpu.sync_copy(
            rows, acc_sh.at[pl.ds(s * ROWS_PER_TILE_ZERO + k * CHUNK, CHUNK)])
    plsc.subcore_barrier()

    def stage(g, carry):
        pltpu.sync_copy(src_hbm.at[s, pl.ds(g * IDXB, IDXB)], sidx)
        pltpu.sync_copy(dst_hbm.at[s, pl.ds(g * IDXB, IDXB)], didx)

        def body(j, carry2):
            @pl.when(c == 0)
            def _():
                pltpu.async_copy(feat_hbm.at[sidx.at[j]], rows, sem).wait()

            @pl.when(c == 1)
            def _():
                pltpu.async_copy(h2_hbm.at[sidx.at[j]], rows, sem).wait()

            pltpu.sync_copy(rows, acc_sh.at[didx.at[j]], add=True)
            return carry2

        lax.fori_loop(0, IDXB, body, 0)
        return carry

    lax.fori_loop(0, NSTAGE, stage, 0)
    plsc.subcore_barrier()

    r0 = s * ROWS_PER_TILE_OUT

    @pl.when(c == 0)
    def _():
        pltpu.sync_copy(acc_sh.at[pl.ds(r0, ROWS_PER_TILE_OUT)],
                        out1.at[pl.ds(r0, ROWS_PER_TILE_OUT)])

    @pl.when(c == 1)
    def _():
        pltpu.sync_copy(acc_sh.at[pl.ds(r0, ROWS_PER_TILE_OUT)],
                        out2.at[pl.ds(r0, ROWS_PER_TILE_OUT)])


def kernel(features, edge_index):
    h2 = _h2(features)
    src = edge_index[0].astype(jnp.int32)
    dst = edge_index[1].astype(jnp.int32)
    pad = EPAD - E
    src_p = jnp.concatenate([src, jnp.zeros((pad,), jnp.int32)])
    dst_p = jnp.concatenate([dst, jnp.full((pad,), N, jnp.int32)])
    src3 = src_p.reshape(NTILES, NCHUNK, CHUNK)
    dst3 = dst_p.reshape(NTILES, NCHUNK, CHUNK)
    zero = jnp.zeros((CHUNK, D), jnp.float32)
    out1, out2 = _sc_scatter(features, h2, src3, dst3, zero)
    return jnp.concatenate([out1[:N], out2[:N]], axis=-1)


# speedup vs baseline: 4.8157x; 1.1856x over previous
import functools

import jax
import jax.numpy as jnp
from jax import lax
from jax.experimental import pallas as pl
from jax.experimental.pallas import tpu as pltpu
from jax.experimental.pallas import tpu_sc as plsc

N = 10000
E = 640000
D = 128
NTILES = 16
CHUNK = 128
NCHUNK = 320
IDXB = 16
NSTAGE = NCHUNK // IDXB
EPAD = NTILES * NCHUNK * CHUNK
NROWS = 10240
ROWS_PER_TILE = NROWS // NTILES

_INV_SQRT2 = 0.7071067811865475


def _h2_body(x_ref, o_ref):
    x = x_ref[...]
    o_ref[...] = (x * x - 1.0) * _INV_SQRT2


def _h2(features):
    return pl.pallas_call(
        _h2_body,
        out_shape=jax.ShapeDtypeStruct((N, D), jnp.float32),
        grid=(10,),
        in_specs=[pl.BlockSpec((N // 10, D), lambda i: (i, 0))],
        out_specs=pl.BlockSpec((N // 10, D), lambda i: (i, 0)),
    )(features)


@functools.partial(
    pl.kernel,
    mesh=plsc.VectorSubcoreMesh(core_axis_name="c", subcore_axis_name="s"),
    out_type=[
        jax.ShapeDtypeStruct((NROWS, D), jnp.float32),
        jax.ShapeDtypeStruct((NROWS, D), jnp.float32),
    ],
    scratch_types=[
        pltpu.VMEM((IDXB, CHUNK), jnp.int32),
        pltpu.VMEM((IDXB, CHUNK), jnp.int32),
        pltpu.VMEM((2, CHUNK, D), jnp.float32),
        pltpu.VMEM_SHARED((NROWS, D), jnp.float32),
        pltpu.SemaphoreType.DMA((2,)),
    ],
)
def _sc_scatter(feat_hbm, h2_hbm, src_hbm, dst_hbm, zero_hbm,
                out1, out2, sidx, didx, rows, acc_sh, gsem):
    c = lax.axis_index("c")
    s = lax.axis_index("s")

    pltpu.sync_copy(zero_hbm, rows.at[0])
    for k in range(ROWS_PER_TILE // CHUNK):
        pltpu.sync_copy(
            rows.at[0], acc_sh.at[pl.ds(s * ROWS_PER_TILE + k * CHUNK, CHUNK)])
    plsc.subcore_barrier()

    def gather_start(jj, b):
        idxrow = sidx.at[jj]

        @pl.when(c == 0)
        def _():
            pltpu.async_copy(feat_hbm.at[idxrow], rows.at[b], gsem.at[b])

        @pl.when(c == 1)
        def _():
            pltpu.async_copy(h2_hbm.at[idxrow], rows.at[b], gsem.at[b])

    def stage(g, carry):
        pltpu.sync_copy(src_hbm.at[s, pl.ds(g * IDXB, IDXB)], sidx)
        pltpu.sync_copy(dst_hbm.at[s, pl.ds(g * IDXB, IDXB)], didx)
        gather_start(0, 0)

        def inner(jj, carry2):
            b = lax.rem(jj, 2)

            @pl.when(jj < IDXB - 1)
            def _():
                gather_start(jj + 1, lax.rem(jj + 1, 2))

            pltpu.make_async_copy(
                feat_hbm.at[sidx.at[jj]], rows.at[b], gsem.at[b]).wait()
            pltpu.sync_copy(rows.at[b], acc_sh.at[didx.at[jj]], add=True)
            return carry2

        lax.fori_loop(0, IDXB, inner, 0)
        return carry

    lax.fori_loop(0, NSTAGE, stage, 0)
    plsc.subcore_barrier()

    r0 = s * ROWS_PER_TILE

    @pl.when(c == 0)
    def _():
        pltpu.sync_copy(acc_sh.at[pl.ds(r0, ROWS_PER_TILE)],
                        out1.at[pl.ds(r0, ROWS_PER_TILE)])

    @pl.when(c == 1)
    def _():
        pltpu.sync_copy(acc_sh.at[pl.ds(r0, ROWS_PER_TILE)],
                        out2.at[pl.ds(r0, ROWS_PER_TILE)])


def kernel(features, edge_index):
    h2 = _h2(features)
    src = edge_index[0].astype(jnp.int32)
    dst = edge_index[1].astype(jnp.int32)
    pad = EPAD - E
    src_p = jnp.concatenate([src, jnp.zeros((pad,), jnp.int32)])
    dst_p = jnp.concatenate([dst, jnp.full((pad,), N, jnp.int32)])
    src3 = src_p.reshape(NTILES, NCHUNK, CHUNK)
    dst3 = dst_p.reshape(NTILES, NCHUNK, CHUNK)
    zero = jnp.zeros((CHUNK, D), jnp.float32)
    out1, out2 = _sc_scatter(features, h2, src3, dst3, zero)
    return jnp.concatenate([out1[:N], out2[:N]], axis=-1)
